# Initial kernel scaffold; baseline (speedup 1.0000x reference)
import jax
import jax.numpy as jnp
from jax.experimental import pallas as pl


def kernel(v, edge_index, W1_0, W1_1, W1_2, b1, W2_0, W2_1, W2_2, b2, W3_0, W3_1, W3_2, b3, Wmu, bmu, Wstd, bstd):
    raise NotImplementedError("write your pallas kernel here")


# speedup vs baseline: 4.2122x; 4.2122x over previous
import functools

import jax
import jax.numpy as jnp
from jax import lax
from jax.experimental import pallas as pl
from jax.experimental.pallas import tpu as pltpu
from jax.experimental.pallas import tpu_sc as plsc

N = 10000
E = 320000
NPAD = 10240
TRASH = NPAD - 1
NC, NS = 2, 16
NW = NC * NS
EB = 128
EPW = 10112
NCHUNK = EPW // EB
EPAD = NW * EPW
ROWS_PER_TILE = NPAD // NS
ZROWS = 64
D = 128

_mesh = lambda: plsc.VectorSubcoreMesh(core_axis_name="c", subcore_axis_name="s")


@functools.partial(
    pl.kernel, mesh=_mesh(),
    out_type=(jax.ShapeDtypeStruct((EPAD,), jnp.int32),
              jax.ShapeDtypeStruct((NC, NPAD), jnp.float32)),
    scratch_types=[
        pltpu.VMEM((EB,), jnp.int32),
        pltpu.VMEM((EB,), jnp.int32),
        pltpu.VMEM((EB,), jnp.int32),
        pltpu.VMEM((EB,), jnp.float32),
        pltpu.VMEM((ROWS_PER_TILE,), jnp.float32),
        pltpu.VMEM_SHARED((NPAD,), jnp.float32),
    ],
)
def _sc_prep(row_hbm, col_hbm, col2_hbm, deg_hbm, rv, cv, c2v, wv, zb, dacc):
    c = lax.axis_index("c")
    s = lax.axis_index("s")
    wid = s * NC + c

    def zfill(i, _):
        zb[pl.ds(i * 16, 16)] = jnp.zeros((16,), jnp.float32)
        return 0
    lax.fori_loop(0, ROWS_PER_TILE // 16, zfill, 0)
    pltpu.sync_copy(zb, dacc.at[pl.ds(s * ROWS_PER_TILE, ROWS_PER_TILE)])
    plsc.subcore_barrier()

    base = wid * EPW

    def body(t, _):
        off = pl.multiple_of(base + t * EB, EB)
        pltpu.sync_copy(row_hbm.at[pl.ds(off, EB)], rv)
        pltpu.sync_copy(col_hbm.at[pl.ds(off, EB)], cv)
        for k in range(EB // 16):
            sl = pl.ds(k * 16, 16)
            r = rv[sl]
            cc = cv[sl]
            m = r == cc
            c2v[sl] = jnp.where(m, jnp.full((16,), TRASH, jnp.int32), cc)
            wv[sl] = jnp.where(m, jnp.zeros((16,), jnp.float32),
                               jnp.ones((16,), jnp.float32))
        pltpu.sync_copy(c2v, col2_hbm.at[pl.ds(off, EB)])
        pltpu.sync_copy(wv, dacc.at[rv], add=True)
        return 0
    lax.fori_loop(0, NCHUNK, body, 0)

    plsc.subcore_barrier()
    pltpu.sync_copy(dacc.at[pl.ds(s * ROWS_PER_TILE, ROWS_PER_TILE)],
                    deg_hbm.at[c, pl.ds(s * ROWS_PER_TILE, ROWS_PER_TILE)])


@functools.partial(
    pl.kernel, mesh=_mesh(),
    out_type=jax.ShapeDtypeStruct((NC, NPAD, D), jnp.float32),
    scratch_types=[
        pltpu.VMEM((EB,), jnp.int32),
        pltpu.VMEM((EB,), jnp.int32),
        pltpu.VMEM((EB, D), jnp.float32),
        pltpu.VMEM((ZROWS, D), jnp.float32),
        pltpu.VMEM_SHARED((NPAD, D), jnp.float32),
        pltpu.SemaphoreType.DMA,
    ],
)
def _sc_prop(x_hbm, row_hbm, col_hbm, out_hbm, ridx, cidx, rows, zbuf, acc, sem):
    c = lax.axis_index("c")
    s = lax.axis_index("s")
    wid = s * NC + c

    def zfill(i, _):
        for k in range(D // 16):
            zbuf[i, pl.ds(k * 16, 16)] = jnp.zeros((16,), jnp.float32)
        return 0
    lax.fori_loop(0, ZROWS, zfill, 0)
    for t in range(ROWS_PER_TILE // ZROWS):
        pltpu.sync_copy(zbuf, acc.at[pl.ds(s * ROWS_PER_TILE + t * ZROWS, ZROWS)])
    plsc.subcore_barrier()

    base = wid * EPW

    def body(t, _):
        off = pl.multiple_of(base + t * EB, EB)
        pltpu.sync_copy(row_hbm.at[pl.ds(off, EB)], ridx)
        pltpu.sync_copy(col_hbm.at[pl.ds(off, EB)], cidx)
        pltpu.async_copy(x_hbm.at[ridx], rows, sem).wait()
        pltpu.sync_copy(rows, acc.at[cidx], add=True)
        return 0
    lax.fori_loop(0, NCHUNK, body, 0)

    plsc.subcore_barrier()
    pltpu.sync_copy(acc.at[pl.ds(s * ROWS_PER_TILE, ROWS_PER_TILE)],
                    out_hbm.at[c, pl.ds(s * ROWS_PER_TILE, ROWS_PER_TILE)])


BN = 256
GRID = NPAD // BN


def _pre_body(degT_ref, v_ref, dinv_ref, xs_ref):
    d = degT_ref[...].sum(axis=1, keepdims=True)
    dinv = jnp.where(d > 0, lax.rsqrt(jnp.maximum(d, 1e-12)), 0.0)
    dinv_ref[...] = dinv
    xs_ref[...] = v_ref[...] * dinv


def _tc_pre(degT, vp):
    return pl.pallas_call(
        _pre_body,
        grid=(GRID,),
        in_specs=[pl.BlockSpec((BN, 2), lambda g: (g, 0)),
                  pl.BlockSpec((BN, 128), lambda g: (g, 0))],
        out_specs=[pl.BlockSpec((BN, 1), lambda g: (g, 0)),
                   pl.BlockSpec((BN, 128), lambda g: (g, 0))],
        out_shape=[jax.ShapeDtypeStruct((NPAD, 1), jnp.float32),
                   jax.ShapeDtypeStruct((NPAD, 128), jnp.float32)],
    )(degT, vp)


def _mid_body(p0_ref, p1_ref, dinv_ref, tx1_ref, tx1s_ref):
    dinv = dinv_ref[...]
    t = -(p0_ref[...] + p1_ref[...]) * dinv
    tx1_ref[...] = t
    tx1s_ref[...] = t * dinv


def _tc_mid(p, dinv):
    blk = lambda d: pl.BlockSpec((BN, d), lambda g: (g, 0))
    return pl.pallas_call(
        _mid_body,
        grid=(GRID,),
        in_specs=[blk(D), blk(D), blk(1)],
        out_specs=[blk(D), blk(D)],
        out_shape=[jax.ShapeDtypeStruct((NPAD, D), jnp.float32),
                   jax.ShapeDtypeStruct((NPAD, D), jnp.float32)],
    )(p[0], p[1], dinv)


def _mid2_body(pl0_ref, pl1_ref, ph0_ref, ph1_ref, dinv_ref,
               tx1_ref, tsl_ref, tsh_ref):
    dinv = dinv_ref[...]
    t_lo = -(pl0_ref[...] + pl1_ref[...]) * dinv
    t_hi = -(ph0_ref[...] + ph1_ref[...]) * dinv
    tx1_ref[:, :D] = t_lo
    tx1_ref[:, D:] = t_hi
    tsl_ref[...] = t_lo * dinv
    tsh_ref[...] = t_hi * dinv


def _tc_mid2(p_lo, p_hi, dinv):
    blk = lambda d: pl.BlockSpec((BN, d), lambda g: (g, 0))
    return pl.pallas_call(
        _mid2_body,
        grid=(GRID,),
        in_specs=[blk(D), blk(D), blk(D), blk(D), blk(1)],
        out_specs=[blk(2 * D), blk(D), blk(D)],
        out_shape=[jax.ShapeDtypeStruct((NPAD, 2 * D), jnp.float32),
                   jax.ShapeDtypeStruct((NPAD, D), jnp.float32),
                   jax.ShapeDtypeStruct((NPAD, D), jnp.float32)],
    )(p_lo[0], p_lo[1], p_hi[0], p_hi[1], dinv)


def _out_body(split_hs, x_ref, tx1_ref, q0_ref, q1_ref, dinv_ref,
              W0_ref, W1_ref, W2_ref, b_ref, *out_refs):
    dinv = dinv_ref[...]
    u = -2.0 * (q0_ref[...] + q1_ref[...]) * dinv
    acc = jnp.dot(x_ref[...], W0_ref[...] - W2_ref[...],
                  preferred_element_type=jnp.float32)
    acc += jnp.dot(tx1_ref[...], W1_ref[...], preferred_element_type=jnp.float32)
    acc += jnp.dot(u, W2_ref[...], preferred_element_type=jnp.float32)
    h = jnp.maximum(acc + b_ref[...], 0.0)
    out_refs[0][...] = h
    hs = h * dinv
    if split_hs:
        out_refs[1][...] = hs[:, :D]
        out_refs[2][...] = hs[:, D:]
    else:
        out_refs[1][...] = hs


def _tc_out(x, tx1, q, dinv, W0, W1, W2, b, Din, Dout):
    blk = lambda d: pl.BlockSpec((BN, d), lambda g: (g, 0))
    wblk = pl.BlockSpec((Din, Dout), lambda g: (0, 0))
    split = Dout == 2 * D
    if split:
        out_specs = [blk(Dout), blk(D), blk(D)]
        out_shape = [jax.ShapeDtypeStruct((NPAD, Dout), jnp.float32),
                     jax.ShapeDtypeStruct((NPAD, D), jnp.float32),
                     jax.ShapeDtypeStruct((NPAD, D), jnp.float32)]
    else:
        out_specs = [blk(Dout), blk(Dout)]
        out_shape = [jax.ShapeDtypeStruct((NPAD, Dout), jnp.float32),
                     jax.ShapeDtypeStruct((NPAD, Dout), jnp.float32)]
    return pl.pallas_call(
        functools.partial(_out_body, split),
        grid=(GRID,),
        in_specs=[blk(Din), blk(Din), blk(Din), blk(Din), blk(1),
                  wblk, wblk, wblk, pl.BlockSpec((1, Dout), lambda g: (0, 0))],
        out_specs=out_specs,
        out_shape=out_shape,
    )(x, tx1, q[0], q[1], dinv, W0, W1, W2, b.reshape(1, Dout))


def _fin_body(x_ref, tx1_ref, ql0_ref, ql1_ref, qh0_ref, qh1_ref, dinv_ref,
              W0_ref, W1_ref, W2_ref, b_ref, Wmu_ref, bmu_ref, Wstd_ref,
              bstd_ref, mu_ref, std_ref):
    dinv = dinv_ref[...]
    u_lo = -2.0 * (ql0_ref[...] + ql1_ref[...]) * dinv
    u_hi = -2.0 * (qh0_ref[...] + qh1_ref[...]) * dinv
    acc = jnp.dot(x_ref[...], W0_ref[...] - W2_ref[...],
                  preferred_element_type=jnp.float32)
    acc += jnp.dot(tx1_ref[...], W1_ref[...], preferred_element_type=jnp.float32)
    acc += jnp.dot(u_lo, W2_ref[:D, :], preferred_element_type=jnp.float32)
    acc += jnp.dot(u_hi, W2_ref[D:, :], preferred_element_type=jnp.float32)
    h = jnp.maximum(acc + b_ref[...], 0.0)
    mu_ref[...] = jnp.dot(h, Wmu_ref[...],
                          preferred_element_type=jnp.float32) + bmu_ref[...]
    std_ref[...] = jnp.dot(h, Wstd_ref[...],
                           preferred_element_type=jnp.float32) + bstd_ref[...]


def _tc_fin(x, tx1, q_lo, q_hi, dinv, W0, W1, W2, b, Wmu, bmu, Wstd, bstd):
    Din, Dout, Dh = 256, 512, 256
    blk = lambda d: pl.BlockSpec((BN, d), lambda g: (g, 0))
    wblk = pl.BlockSpec((Din, Dout), lambda g: (0, 0))
    hblk = pl.BlockSpec((Dout, Dh), lambda g: (0, 0))
    bblk = lambda d: pl.BlockSpec((1, d), lambda g: (0, 0))
    return pl.pallas_call(
        _fin_body,
        grid=(GRID,),
        in_specs=[blk(Din), blk(Din), blk(D), blk(D), blk(D), blk(D), blk(1),
                  wblk, wblk, wblk, bblk(Dout),
                  hblk, bblk(Dh), hblk, bblk(Dh)],
        out_specs=[blk(Dh), blk(Dh)],
        out_shape=[jax.ShapeDtypeStruct((NPAD, Dh), jnp.float32),
                   jax.ShapeDtypeStruct((NPAD, Dh), jnp.float32)],
    )(x, tx1, q_lo[0], q_lo[1], q_hi[0], q_hi[1], dinv,
      W0, W1, W2, b.reshape(1, Dout),
      Wmu, bmu.reshape(1, Dh), Wstd, bstd.reshape(1, Dh))


def kernel(v, edge_index, W1_0, W1_1, W1_2, b1, W2_0, W2_1, W2_2, b2,
           W3_0, W3_1, W3_2, b3, Wmu, bmu, Wstd, bstd):
    row = edge_index[0].astype(jnp.int32)
    col = edge_index[1].astype(jnp.int32)
    rowp = jnp.pad(row, (0, EPAD - E))
    colp = jnp.pad(col, (0, EPAD - E))
    vp = jnp.pad(v, ((0, NPAD - N), (0, 0)))

    col2, degp = _sc_prep(rowp, colp)
    dinv, xs = _tc_pre(degp.T, vp)

    def layer(x, xs, W0, W1, W2, b, Din, Dout):
        p = _sc_prop(xs, rowp, col2)
        tx1, tx1s = _tc_mid(p, dinv)
        q = _sc_prop(tx1s, rowp, col2)
        return _tc_out(x, tx1, q, dinv, W0, W1, W2, b, Din, Dout)

    h1, h1s = layer(vp, xs, W1_0, W1_1, W1_2, b1, 128, 128)
    h2, h2s_lo, h2s_hi = layer(h1, h1s, W2_0, W2_1, W2_2, b2, 128, 256)

    p_lo = _sc_prop(h2s_lo, rowp, col2)
    p_hi = _sc_prop(h2s_hi, rowp, col2)
    tx1, tsl, tsh = _tc_mid2(p_lo, p_hi, dinv)
    q_lo = _sc_prop(tsl, rowp, col2)
    q_hi = _sc_prop(tsh, rowp, col2)
    mu, std = _tc_fin(h2, tx1, q_lo, q_hi, dinv, W3_0, W3_1, W3_2, b3,
                      Wmu, bmu, Wstd, bstd)
    return mu[:N], std[:N]
